# baseline (speedup 1.0000x reference)
import functools

import jax
import jax.numpy as jnp
from jax import lax
from jax.experimental import pallas as pl
from jax.experimental.pallas import tpu as pltpu
from jax.experimental.pallas import tpu_sc as plsc

N = 10000
NP = 10240
D = 128
E = 640000
C = 2
K = 128
NC, NS = 2, 16
NW = NC * NS
CB = 16
NB = 10
CHUNKS = CB * NB
EP = CHUNKS * NW * K
EPW = CHUNKS * K
RPT = NP // NS

_mesh = plsc.VectorSubcoreMesh(core_axis_name="c", subcore_axis_name="s",
                               num_cores=NC, num_subcores=NS)
_f32 = jnp.float32


def _fill2d(ref, rows, cols, val):
    def row(i, _):
        def col(j, _):
            ref[i, pl.ds(j * 16, 16)] = jnp.full((16,), val, _f32)
            return 0
        return lax.fori_loop(0, cols // 16, col, 0)
    lax.fori_loop(0, rows, row, 0)


def _deg_body(dst_hbm, out_hbm, idx_v, ones_v, zb_v, acc_sh):
    c = lax.axis_index("c")
    s = lax.axis_index("s")
    w = s * NC + c
    _fill2d(ones_v, K, D, 1.0)
    _fill2d(zb_v, 64, D, 0.0)

    def z(i, _):
        pltpu.sync_copy(zb_v, acc_sh.at[pl.ds(s * RPT + i * 64, 64)])
        return 0
    lax.fori_loop(0, RPT // 64, z, 0)
    plsc.subcore_barrier()

    def step(k, _):
        base = w * EPW + k * K
        pltpu.sync_copy(dst_hbm.at[pl.ds(base, K)], idx_v)
        pltpu.sync_copy(ones_v, acc_sh.at[idx_v], add=True)
        return 0
    lax.fori_loop(0, CHUNKS, step, 0)
    plsc.subcore_barrier()
    pltpu.sync_copy(acc_sh.at[pl.ds(s * RPT, RPT)],
                    out_hbm.at[c, pl.ds(s * RPT, RPT)])


_deg_call = functools.partial(
    pl.kernel,
    out_type=jax.ShapeDtypeStruct((NC, NP, D), _f32),
    mesh=_mesh,
    scratch_types=[
        pltpu.VMEM((K,), jnp.int32),
        pltpu.VMEM((K, D), _f32),
        pltpu.VMEM((64, D), _f32),
        pltpu.VMEM_SHARED((NP, D), _f32),
    ],
)(_deg_body)


def _agg_body(g_hbm, src_hbm, dst_hbm, out_hbm,
              sidx_v, didx_v, rows0_v, rows1_v, zb_v, acc_sh,
              sem0, sem1):
    c = lax.axis_index("c")
    s = lax.axis_index("s")
    w = s * NC + c
    _fill2d(zb_v, 32, D, 0.0)

    def z(i, _):
        pltpu.sync_copy(zb_v, acc_sh.at[pl.ds(s * RPT + i * 32, 32)])
        return 0
    lax.fori_loop(0, RPT // 32, z, 0)
    plsc.subcore_barrier()

    def gstart(j, rv, sem):
        pltpu.async_copy(g_hbm.at[sidx_v.at[j]], rv, sem)

    def gwait(j, rv, sem):
        pltpu.make_async_copy(g_hbm.at[sidx_v.at[j]], rv, sem).wait()

    def scat(j, rv):
        pltpu.sync_copy(rv, acc_sh.at[didx_v.at[j]], add=True)

    def block(b, _):
        pltpu.sync_copy(src_hbm.at[w, pl.ds(b * CB, CB)], sidx_v)
        pltpu.sync_copy(dst_hbm.at[w, pl.ds(b * CB, CB)], didx_v)
        gstart(0, rows0_v, sem0)

        def pair(t, _):
            j0 = 2 * t
            gstart(j0 + 1, rows1_v, sem1)
            gwait(j0, rows0_v, sem0)
            scat(j0, rows0_v)
            gstart(j0 + 2, rows0_v, sem0)
            gwait(j0 + 1, rows1_v, sem1)
            scat(j0 + 1, rows1_v)
            return 0
        lax.fori_loop(0, CB // 2 - 1, pair, 0)
        gstart(CB - 1, rows1_v, sem1)
        gwait(CB - 2, rows0_v, sem0)
        scat(CB - 2, rows0_v)
        gwait(CB - 1, rows1_v, sem1)
        scat(CB - 1, rows1_v)
        return 0
    lax.fori_loop(0, NB, block, 0)
    plsc.subcore_barrier()
    pltpu.sync_copy(acc_sh.at[pl.ds(s * RPT, RPT)],
                    out_hbm.at[c, pl.ds(s * RPT, RPT)])


_agg_call = functools.partial(
    pl.kernel,
    out_type=jax.ShapeDtypeStruct((NC, NP, D), _f32),
    mesh=_mesh,
    scratch_types=[
        pltpu.VMEM((CB, K), jnp.int32),
        pltpu.VMEM((CB, K), jnp.int32),
        pltpu.VMEM((K, D), _f32),
        pltpu.VMEM((K, D), _f32),
        pltpu.VMEM((32, D), _f32),
        pltpu.VMEM_SHARED((NP, D), _f32),
        pltpu.SemaphoreType.DMA,
        pltpu.SemaphoreType.DMA,
    ],
)(_agg_body)


def _head_body(a_hbm, b_hbm, src_hbm, dst_hbm, out_hbm,
               sidx_v, didx_v, ra_v, rb_v, iota_v, stage_sh, sem_a, sem_b):
    c = lax.axis_index("c")
    s = lax.axis_index("s")
    w = s * NC + c

    def f(j, _):
        iota_v[pl.ds(j * 16, 16)] = lax.iota(jnp.int32, 16) + (s * K + j * 16)
        return 0
    lax.fori_loop(0, K // 16, f, 0)

    def step(k, _):
        base = w * EPW + k * K
        pltpu.sync_copy(src_hbm.at[pl.ds(base, K)], sidx_v)
        pltpu.sync_copy(dst_hbm.at[pl.ds(base, K)], didx_v)
        cp_a = pltpu.async_copy(a_hbm.at[sidx_v], ra_v, sem_a)
        cp_b = pltpu.async_copy(b_hbm.at[didx_v], rb_v, sem_b)
        cp_a.wait()
        cp_b.wait()
        pltpu.sync_copy(ra_v, stage_sh.at[pl.ds(s * K, K)])
        pltpu.sync_copy(rb_v, stage_sh.at[iota_v], add=True)
        pltpu.sync_copy(stage_sh.at[pl.ds(s * K, K)], out_hbm.at[pl.ds(base, K)])
        return 0
    lax.fori_loop(0, CHUNKS, step, 0)


_head_call = functools.partial(
    pl.kernel,
    out_type=jax.ShapeDtypeStruct((EP, D), _f32),
    mesh=_mesh,
    scratch_types=[
        pltpu.VMEM((K,), jnp.int32),
        pltpu.VMEM((K,), jnp.int32),
        pltpu.VMEM((K, D), _f32),
        pltpu.VMEM((K, D), _f32),
        pltpu.VMEM((K,), jnp.int32),
        pltpu.VMEM_SHARED((NS * K, D), _f32),
        pltpu.SemaphoreType.DMA,
        pltpu.SemaphoreType.DMA,
    ],
)(_head_body)


def _dis(degp_ref):
    deg = degp_ref[0, :, 0:1] + degp_ref[1, :, 0:1] + 1.0
    return lax.rsqrt(deg)


def _s1_body(x_ref, w1_ref, degp_ref, g1_ref):
    dis = _dis(degp_ref)
    hw = jnp.dot(x_ref[...], w1_ref[...], preferred_element_type=_f32)
    g1_ref[...] = hw * dis


def _s2_body(agg_ref, x_ref, w1_ref, b1_ref, w2_ref, degp_ref, g2_ref, hw2_ref):
    dis = _dis(degp_ref)
    hw1 = jnp.dot(x_ref[...], w1_ref[...], preferred_element_type=_f32)
    h1 = jnp.maximum(dis * (agg_ref[0] + agg_ref[1]) + dis * dis * hw1
                     + b1_ref[...], 0.0)
    hw2 = jnp.dot(h1, w2_ref[...], preferred_element_type=_f32)
    hw2_ref[...] = hw2
    g2_ref[...] = hw2 * dis


def _s3_body(agg_ref, hw2_ref, b2_ref, w1a_ref, w1b_ref, l1b_ref, degp_ref,
             a_ref, b_ref):
    dis = _dis(degp_ref)
    h2 = jnp.maximum(dis * (agg_ref[0] + agg_ref[1]) + dis * dis * hw2_ref[...]
                     + b2_ref[...], 0.0)
    a_ref[...] = jnp.dot(h2, w1a_ref[...], preferred_element_type=_f32) + l1b_ref[...]
    b_ref[...] = jnp.dot(h2, w1b_ref[...], preferred_element_type=_f32)


BE = 4096


def _s4_body(s_ref, wf_ref, bf_ref, o_ref):
    t = jnp.maximum(s_ref[...], 0.0)
    z = jnp.dot(t, wf_ref[...], preferred_element_type=_f32) + bf_ref[...]
    m = jnp.max(z, axis=1, keepdims=True)
    o_ref[...] = z - m - jnp.log(jnp.sum(jnp.exp(z - m), axis=1, keepdims=True))


def kernel(x, edge_index, W1, b1, W2, b2, lin1_W, lin1_b, linf_W, linf_b):
    src = edge_index[0]
    dst = edge_index[1]
    pad = jnp.full((EP - E,), NP - 1, jnp.int32)
    src_p = jnp.concatenate([src, pad])
    dst_p = jnp.concatenate([dst, pad])
    src_w = src_p.reshape(NW, CHUNKS, K)
    dst_w = dst_p.reshape(NW, CHUNKS, K)
    x_p = jnp.concatenate([x, jnp.zeros((NP - N, x.shape[1]), _f32)])

    degp = _deg_call(dst_p)

    g1 = pl.pallas_call(
        _s1_body,
        out_shape=jax.ShapeDtypeStruct((NP, D), _f32),
    )(x_p, W1, degp)

    agg1 = _agg_call(g1, src_w, dst_w)

    g2, hw2 = pl.pallas_call(
        _s2_body,
        out_shape=[jax.ShapeDtypeStruct((NP, D), _f32),
                   jax.ShapeDtypeStruct((NP, D), _f32)],
    )(agg1, x_p, W1, b1.reshape(1, D), W2, degp)

    agg2 = _agg_call(g2, src_w, dst_w)

    A, B = pl.pallas_call(
        _s3_body,
        out_shape=[jax.ShapeDtypeStruct((NP, D), _f32),
                   jax.ShapeDtypeStruct((NP, D), _f32)],
    )(agg2, hw2, b2.reshape(1, D), lin1_W[:D], lin1_W[D:], lin1_b.reshape(1, D),
      degp)

    s = _head_call(A, B, src_p, dst_p)

    outp = pl.pallas_call(
        _s4_body,
        grid=(EP // BE,),
        in_specs=[
            pl.BlockSpec((BE, D), lambda i: (i, 0)),
            pl.BlockSpec((D, C), lambda i: (0, 0)),
            pl.BlockSpec((1, C), lambda i: (0, 0)),
        ],
        out_specs=pl.BlockSpec((BE, C), lambda i: (i, 0)),
        out_shape=jax.ShapeDtypeStruct((EP, C), _f32),
    )(s, linf_W, linf_b.reshape(1, C))

    return lax.slice(outp, (0, 0), (E, C))


# speedup vs baseline: 2.0008x; 2.0008x over previous
import functools

import jax
import jax.numpy as jnp
from jax import lax
from jax.experimental import pallas as pl
from jax.experimental.pallas import tpu as pltpu
from jax.experimental.pallas import tpu_sc as plsc

N = 10000
NP = 10240
D = 128
E = 640000
C = 2
K = 128
NC, NS = 2, 16
NW = NC * NS
CB = 16
NB = 10
CHUNKS = CB * NB
EP = CHUNKS * NW * K
EPW = CHUNKS * K
RPT = NP // NS

_mesh = plsc.VectorSubcoreMesh(core_axis_name="c", subcore_axis_name="s",
                               num_cores=NC, num_subcores=NS)
_f32 = jnp.float32


def _fill2d(ref, rows, cols, val):
    def row(i, _):
        def col(j, _):
            ref[i, pl.ds(j * 16, 16)] = jnp.full((16,), val, _f32)
            return 0
        return lax.fori_loop(0, cols // 16, col, 0)
    lax.fori_loop(0, rows, row, 0)


def _deg_body(dst_hbm, out_hbm, idx_v, ones_v, zb_v, acc_sh):
    c = lax.axis_index("c")
    s = lax.axis_index("s")
    w = s * NC + c
    _fill2d(ones_v, K, D, 1.0)
    _fill2d(zb_v, 64, D, 0.0)

    def z(i, _):
        pltpu.sync_copy(zb_v, acc_sh.at[pl.ds(s * RPT + i * 64, 64)])
        return 0
    lax.fori_loop(0, RPT // 64, z, 0)
    plsc.subcore_barrier()

    def step(k, _):
        base = w * EPW + k * K
        pltpu.sync_copy(dst_hbm.at[pl.ds(base, K)], idx_v)
        pltpu.sync_copy(ones_v, acc_sh.at[idx_v], add=True)
        return 0
    lax.fori_loop(0, CHUNKS, step, 0)
    plsc.subcore_barrier()
    pltpu.sync_copy(acc_sh.at[pl.ds(s * RPT, RPT)],
                    out_hbm.at[c, pl.ds(s * RPT, RPT)])


_deg_call = functools.partial(
    pl.kernel,
    out_type=jax.ShapeDtypeStruct((NC, NP, D), _f32),
    mesh=_mesh,
    scratch_types=[
        pltpu.VMEM((K,), jnp.int32),
        pltpu.VMEM((K, D), _f32),
        pltpu.VMEM((64, D), _f32),
        pltpu.VMEM_SHARED((NP, D), _f32),
    ],
)(_deg_body)


def _agg_body(g_hbm, src_hbm, dst_hbm, out_hbm,
              sidx_v, didx_v, rows0_v, rows1_v, zb_v, acc_sh,
              sem0, sem1):
    c = lax.axis_index("c")
    s = lax.axis_index("s")
    w = s * NC + c
    _fill2d(zb_v, 32, D, 0.0)

    def z(i, _):
        pltpu.sync_copy(zb_v, acc_sh.at[pl.ds(s * RPT + i * 32, 32)])
        return 0
    lax.fori_loop(0, RPT // 32, z, 0)
    plsc.subcore_barrier()

    def gstart(j, rv, sem):
        pltpu.async_copy(g_hbm.at[sidx_v.at[j]], rv, sem)

    def gwait(j, rv, sem):
        pltpu.make_async_copy(g_hbm.at[sidx_v.at[j]], rv, sem).wait()

    def scat(j, rv):
        pltpu.sync_copy(rv, acc_sh.at[didx_v.at[j]], add=True)

    def block(b, _):
        pltpu.sync_copy(src_hbm.at[w, pl.ds(b * CB, CB)], sidx_v)
        pltpu.sync_copy(dst_hbm.at[w, pl.ds(b * CB, CB)], didx_v)
        gstart(0, rows0_v, sem0)

        def pair(t, _):
            j0 = 2 * t
            gstart(j0 + 1, rows1_v, sem1)
            gwait(j0, rows0_v, sem0)
            scat(j0, rows0_v)
            gstart(j0 + 2, rows0_v, sem0)
            gwait(j0 + 1, rows1_v, sem1)
            scat(j0 + 1, rows1_v)
            return 0
        lax.fori_loop(0, CB // 2 - 1, pair, 0)
        gstart(CB - 1, rows1_v, sem1)
        gwait(CB - 2, rows0_v, sem0)
        scat(CB - 2, rows0_v)
        gwait(CB - 1, rows1_v, sem1)
        scat(CB - 1, rows1_v)
        return 0
    lax.fori_loop(0, NB, block, 0)
    plsc.subcore_barrier()
    pltpu.sync_copy(acc_sh.at[pl.ds(s * RPT, RPT)],
                    out_hbm.at[c, pl.ds(s * RPT, RPT)])


_agg_call = functools.partial(
    pl.kernel,
    out_type=jax.ShapeDtypeStruct((NC, NP, D), _f32),
    mesh=_mesh,
    scratch_types=[
        pltpu.VMEM((CB, K), jnp.int32),
        pltpu.VMEM((CB, K), jnp.int32),
        pltpu.VMEM((K, D), _f32),
        pltpu.VMEM((K, D), _f32),
        pltpu.VMEM((32, D), _f32),
        pltpu.VMEM_SHARED((NP, D), _f32),
        pltpu.SemaphoreType.DMA,
        pltpu.SemaphoreType.DMA,
    ],
)(_agg_body)


def _head_body(a_hbm, b_hbm, src_hbm, dst_hbm, out_hbm,
               sidx_v, didx_v, ra_v, rb_v, iota_v, stage_sh, sem_a, sem_b):
    c = lax.axis_index("c")
    s = lax.axis_index("s")
    w = s * NC + c

    def f(j, _):
        iota_v[pl.ds(j * 16, 16)] = lax.iota(jnp.int32, 16) + (s * K + j * 16)
        return 0
    lax.fori_loop(0, K // 16, f, 0)

    def step(k, _):
        base = w * EPW + k * K
        pltpu.sync_copy(src_hbm.at[pl.ds(base, K)], sidx_v)
        pltpu.sync_copy(dst_hbm.at[pl.ds(base, K)], didx_v)
        cp_a = pltpu.async_copy(a_hbm.at[sidx_v], ra_v, sem_a)
        cp_b = pltpu.async_copy(b_hbm.at[didx_v], rb_v, sem_b)
        cp_a.wait()
        cp_b.wait()
        pltpu.sync_copy(ra_v, stage_sh.at[pl.ds(s * K, K)])
        pltpu.sync_copy(rb_v, stage_sh.at[iota_v], add=True)
        pltpu.sync_copy(stage_sh.at[pl.ds(s * K, K)], out_hbm.at[pl.ds(base, K)])
        return 0
    lax.fori_loop(0, CHUNKS, step, 0)


_head_call = functools.partial(
    pl.kernel,
    out_type=jax.ShapeDtypeStruct((EP, D), _f32),
    mesh=_mesh,
    scratch_types=[
        pltpu.VMEM((K,), jnp.int32),
        pltpu.VMEM((K,), jnp.int32),
        pltpu.VMEM((K, D), _f32),
        pltpu.VMEM((K, D), _f32),
        pltpu.VMEM((K,), jnp.int32),
        pltpu.VMEM_SHARED((NS * K, D), _f32),
        pltpu.SemaphoreType.DMA,
        pltpu.SemaphoreType.DMA,
    ],
)(_head_body)


def _dis(degp_ref):
    deg = degp_ref[0, :, 0:1] + degp_ref[1, :, 0:1] + 1.0
    return lax.rsqrt(deg)


def _s1_body(x_ref, w1_ref, degp_ref, g1_ref):
    dis = _dis(degp_ref)
    hw = jnp.dot(x_ref[...], w1_ref[...], preferred_element_type=_f32)
    g1_ref[...] = hw * dis


def _s2_body(agg_ref, x_ref, w1_ref, b1_ref, w2_ref, degp_ref, g2_ref, hw2_ref):
    dis = _dis(degp_ref)
    hw1 = jnp.dot(x_ref[...], w1_ref[...], preferred_element_type=_f32)
    h1 = jnp.maximum(dis * (agg_ref[0] + agg_ref[1]) + dis * dis * hw1
                     + b1_ref[...], 0.0)
    hw2 = jnp.dot(h1, w2_ref[...], preferred_element_type=_f32)
    hw2_ref[...] = hw2
    g2_ref[...] = hw2 * dis


def _s3_body(agg_ref, hw2_ref, b2_ref, w1a_ref, w1b_ref, l1b_ref, degp_ref,
             a_ref, b_ref):
    dis = _dis(degp_ref)
    h2 = jnp.maximum(dis * (agg_ref[0] + agg_ref[1]) + dis * dis * hw2_ref[...]
                     + b2_ref[...], 0.0)
    a_ref[...] = jnp.dot(h2, w1a_ref[...], preferred_element_type=_f32) + l1b_ref[...]
    b_ref[...] = jnp.dot(h2, w1b_ref[...], preferred_element_type=_f32)


BE = 4096


def _s4_body(s_ref, wf_ref, bf_ref, o_ref):
    t = jnp.maximum(s_ref[...], 0.0)
    z = jnp.dot(t, wf_ref[...], preferred_element_type=_f32) + bf_ref[...]
    m = jnp.max(z, axis=1, keepdims=True)
    o_ref[...] = z - m - jnp.log(jnp.sum(jnp.exp(z - m), axis=1, keepdims=True))


def kernel(x, edge_index, W1, b1, W2, b2, lin1_W, lin1_b, linf_W, linf_b):
    src = edge_index[0]
    dst = edge_index[1]
    pad = (N + jnp.arange(EP - E, dtype=jnp.int32) % (NP - N)).astype(jnp.int32)
    src_p = jnp.concatenate([src, pad])
    dst_p = jnp.concatenate([dst, pad])
    src_w = src_p.reshape(NW, CHUNKS, K)
    dst_w = dst_p.reshape(NW, CHUNKS, K)
    x_p = jnp.concatenate([x, jnp.zeros((NP - N, x.shape[1]), _f32)])

    degp = _deg_call(dst_p)

    g1 = pl.pallas_call(
        _s1_body,
        out_shape=jax.ShapeDtypeStruct((NP, D), _f32),
    )(x_p, W1, degp)

    agg1 = _agg_call(g1, src_w, dst_w)

    g2, hw2 = pl.pallas_call(
        _s2_body,
        out_shape=[jax.ShapeDtypeStruct((NP, D), _f32),
                   jax.ShapeDtypeStruct((NP, D), _f32)],
    )(agg1, x_p, W1, b1.reshape(1, D), W2, degp)

    agg2 = _agg_call(g2, src_w, dst_w)

    A, B = pl.pallas_call(
        _s3_body,
        out_shape=[jax.ShapeDtypeStruct((NP, D), _f32),
                   jax.ShapeDtypeStruct((NP, D), _f32)],
    )(agg2, hw2, b2.reshape(1, D), lin1_W[:D], lin1_W[D:], lin1_b.reshape(1, D),
      degp)

    s = _head_call(A, B, src_p, dst_p)

    outp = pl.pallas_call(
        _s4_body,
        grid=(EP // BE,),
        in_specs=[
            pl.BlockSpec((BE, D), lambda i: (i, 0)),
            pl.BlockSpec((D, C), lambda i: (0, 0)),
            pl.BlockSpec((1, C), lambda i: (0, 0)),
        ],
        out_specs=pl.BlockSpec((BE, C), lambda i: (i, 0)),
        out_shape=jax.ShapeDtypeStruct((EP, C), _f32),
    )(s, linf_W, linf_b.reshape(1, C))

    return lax.slice(outp, (0, 0), (E, C))


# speedup vs baseline: 2.4624x; 1.2307x over previous
import functools

import jax
import jax.numpy as jnp
from jax import lax
from jax.experimental import pallas as pl
from jax.experimental.pallas import tpu as pltpu
from jax.experimental.pallas import tpu_sc as plsc

N = 10000
NP = 10240
D = 128
E = 640000
C = 2
K = 128
NC, NS = 2, 16
NW = NC * NS
CB = 16
NB = 10
CHUNKS = CB * NB
EP = CHUNKS * NW * K
EPW = CHUNKS * K
RPT = NP // NS

_mesh = plsc.VectorSubcoreMesh(core_axis_name="c", subcore_axis_name="s",
                               num_cores=NC, num_subcores=NS)
_f32 = jnp.float32


def _fill2d(ref, rows, cols, val):
    def row(i, _):
        def col(j, _):
            ref[i, pl.ds(j * 16, 16)] = jnp.full((16,), val, _f32)
            return 0
        return lax.fori_loop(0, cols // 16, col, 0)
    lax.fori_loop(0, rows, row, 0)


def _deg_body(dst_hbm, out_hbm, idx_v, ones_v, zb_v, acc_sh):
    c = lax.axis_index("c")
    s = lax.axis_index("s")
    w = s * NC + c
    _fill2d(ones_v, K, D, 1.0)
    _fill2d(zb_v, 64, D, 0.0)

    def z(i, _):
        pltpu.sync_copy(zb_v, acc_sh.at[pl.ds(s * RPT + i * 64, 64)])
        return 0
    lax.fori_loop(0, RPT // 64, z, 0)
    plsc.subcore_barrier()

    def step(k, _):
        base = w * EPW + k * K
        pltpu.sync_copy(dst_hbm.at[pl.ds(base, K)], idx_v)
        pltpu.sync_copy(ones_v, acc_sh.at[idx_v], add=True)
        return 0
    lax.fori_loop(0, CHUNKS, step, 0)
    plsc.subcore_barrier()
    pltpu.sync_copy(acc_sh.at[pl.ds(s * RPT, RPT)],
                    out_hbm.at[c, pl.ds(s * RPT, RPT)])


_deg_call = functools.partial(
    pl.kernel,
    out_type=jax.ShapeDtypeStruct((NC, NP, D), _f32),
    mesh=_mesh,
    scratch_types=[
        pltpu.VMEM((K,), jnp.int32),
        pltpu.VMEM((K, D), _f32),
        pltpu.VMEM((64, D), _f32),
        pltpu.VMEM_SHARED((NP, D), _f32),
    ],
)(_deg_body)


def _agg_body(g_hbm, src_hbm, dst_hbm, out_hbm,
              sidx_v, didx_v, rows0_v, rows1_v, zb_v, acc_sh,
              sem0, sem1):
    c = lax.axis_index("c")
    s = lax.axis_index("s")
    w = s * NC + c
    _fill2d(zb_v, 32, D, 0.0)

    def z(i, _):
        pltpu.sync_copy(zb_v, acc_sh.at[pl.ds(s * RPT + i * 32, 32)])
        return 0
    lax.fori_loop(0, RPT // 32, z, 0)
    plsc.subcore_barrier()

    def gstart(j, rv, sem):
        pltpu.async_copy(g_hbm.at[sidx_v.at[j]], rv, sem)

    def gwait(j, rv, sem):
        pltpu.make_async_copy(g_hbm.at[sidx_v.at[j]], rv, sem).wait()

    def scat(j, rv):
        pltpu.sync_copy(rv, acc_sh.at[didx_v.at[j]], add=True)

    def block(b, _):
        pltpu.sync_copy(src_hbm.at[w, pl.ds(b * CB, CB)], sidx_v)
        pltpu.sync_copy(dst_hbm.at[w, pl.ds(b * CB, CB)], didx_v)
        gstart(0, rows0_v, sem0)

        def pair(t, _):
            j0 = 2 * t
            gstart(j0 + 1, rows1_v, sem1)
            gwait(j0, rows0_v, sem0)
            scat(j0, rows0_v)
            gstart(j0 + 2, rows0_v, sem0)
            gwait(j0 + 1, rows1_v, sem1)
            scat(j0 + 1, rows1_v)
            return 0
        lax.fori_loop(0, CB // 2 - 1, pair, 0)
        gstart(CB - 1, rows1_v, sem1)
        gwait(CB - 2, rows0_v, sem0)
        scat(CB - 2, rows0_v)
        gwait(CB - 1, rows1_v, sem1)
        scat(CB - 1, rows1_v)
        return 0
    lax.fori_loop(0, NB, block, 0)
    plsc.subcore_barrier()
    pltpu.sync_copy(acc_sh.at[pl.ds(s * RPT, RPT)],
                    out_hbm.at[c, pl.ds(s * RPT, RPT)])


_agg_call = functools.partial(
    pl.kernel,
    out_type=jax.ShapeDtypeStruct((NC, NP, D), _f32),
    mesh=_mesh,
    scratch_types=[
        pltpu.VMEM((CB, K), jnp.int32),
        pltpu.VMEM((CB, K), jnp.int32),
        pltpu.VMEM((K, D), _f32),
        pltpu.VMEM((K, D), _f32),
        pltpu.VMEM((32, D), _f32),
        pltpu.VMEM_SHARED((NP, D), _f32),
        pltpu.SemaphoreType.DMA,
        pltpu.SemaphoreType.DMA,
    ],
)(_agg_body)


def _head_body(a_hbm, b_hbm, src_hbm, dst_hbm, out_hbm,
               sidx_v, didx_v, ra0_v, ra1_v, rb0_v, rb1_v, t0_v, t1_v,
               semg0, semg1, semo0, semo1):
    c = lax.axis_index("c")
    s = lax.axis_index("s")
    w = s * NC + c
    semg = (semg0, semg1)
    semo = (semo0, semo1)
    ras = (ra0_v, ra1_v)
    rbs = (rb0_v, rb1_v)
    ts = (t0_v, t1_v)

    def gstart(j, p):
        pltpu.async_copy(a_hbm.at[sidx_v.at[j]], ras[p], semg[p])
        pltpu.async_copy(b_hbm.at[didx_v.at[j]], rbs[p], semg[p])

    def gwait(j, p):
        pltpu.make_async_copy(a_hbm.at[sidx_v.at[j]], ras[p], semg[p]).wait()
        pltpu.make_async_copy(b_hbm.at[didx_v.at[j]], rbs[p], semg[p]).wait()

    def combine(p):
        def row(i, _):
            for g in range(D // 16):
                sl = pl.ds(g * 16, 16)
                ts[p][i, sl] = jnp.maximum(ras[p][i, sl] + rbs[p][i, sl], 0.0)
            return 0
        lax.fori_loop(0, K, row, 0)

    def ostart(p, base):
        pltpu.async_copy(ts[p], out_hbm.at[pl.ds(base, K)], semo[p])

    def owait(p):
        pltpu.make_async_copy(ts[p], out_hbm.at[pl.ds(w * EPW, K)],
                              semo[p]).wait()

    def block(b, _):
        pltpu.sync_copy(src_hbm.at[w, pl.ds(b * CB, CB)], sidx_v)
        pltpu.sync_copy(dst_hbm.at[w, pl.ds(b * CB, CB)], didx_v)
        gstart(0, 0)
        gstart(1, 1)

        def pair(t, _):
            for dp in (0, 1):
                j = 2 * t + dp
                gwait(j, dp)

                @pl.when((t > 0) | (b > 0))
                def _(dp=dp):
                    owait(dp)
                combine(dp)
                ostart(dp, w * EPW + (b * CB + j) * K)

                @pl.when(j + 2 < CB)
                def _(j=j, dp=dp):
                    gstart(j + 2, dp)
            return 0
        lax.fori_loop(0, CB // 2, pair, 0)
        return 0
    lax.fori_loop(0, NB, block, 0)
    owait(0)
    owait(1)


_head_call = functools.partial(
    pl.kernel,
    out_type=jax.ShapeDtypeStruct((EP, D), _f32),
    mesh=_mesh,
    scratch_types=[
        pltpu.VMEM((CB, K), jnp.int32),
        pltpu.VMEM((CB, K), jnp.int32),
        pltpu.VMEM((K, D), _f32),
        pltpu.VMEM((K, D), _f32),
        pltpu.VMEM((K, D), _f32),
        pltpu.VMEM((K, D), _f32),
        pltpu.VMEM((K, D), _f32),
        pltpu.VMEM((K, D), _f32),
        pltpu.SemaphoreType.DMA,
        pltpu.SemaphoreType.DMA,
        pltpu.SemaphoreType.DMA,
        pltpu.SemaphoreType.DMA,
    ],
)(_head_body)


def _dis(degp_ref):
    deg = degp_ref[0, :, 0:1] + degp_ref[1, :, 0:1] + 1.0
    return lax.rsqrt(deg)


def _s1_body(x_ref, w1_ref, degp_ref, g1_ref):
    dis = _dis(degp_ref)
    hw = jnp.dot(x_ref[...], w1_ref[...], preferred_element_type=_f32)
    g1_ref[...] = hw * dis


def _s2_body(agg_ref, x_ref, w1_ref, b1_ref, w2_ref, degp_ref, g2_ref, hw2_ref):
    dis = _dis(degp_ref)
    hw1 = jnp.dot(x_ref[...], w1_ref[...], preferred_element_type=_f32)
    h1 = jnp.maximum(dis * (agg_ref[0] + agg_ref[1]) + dis * dis * hw1
                     + b1_ref[...], 0.0)
    hw2 = jnp.dot(h1, w2_ref[...], preferred_element_type=_f32)
    hw2_ref[...] = hw2
    g2_ref[...] = hw2 * dis


def _s3_body(agg_ref, hw2_ref, b2_ref, w1a_ref, w1b_ref, l1b_ref, degp_ref,
             a_ref, b_ref):
    dis = _dis(degp_ref)
    h2 = jnp.maximum(dis * (agg_ref[0] + agg_ref[1]) + dis * dis * hw2_ref[...]
                     + b2_ref[...], 0.0)
    a_ref[...] = jnp.dot(h2, w1a_ref[...], preferred_element_type=_f32) + l1b_ref[...]
    b_ref[...] = jnp.dot(h2, w1b_ref[...], preferred_element_type=_f32)


BE = 4096


def _s4_body(s_ref, wf_ref, bf_ref, o_ref):
    t = s_ref[...]
    z = jnp.dot(t, wf_ref[...], preferred_element_type=_f32) + bf_ref[...]
    m = jnp.max(z, axis=1, keepdims=True)
    o_ref[...] = z - m - jnp.log(jnp.sum(jnp.exp(z - m), axis=1, keepdims=True))


def kernel(x, edge_index, W1, b1, W2, b2, lin1_W, lin1_b, linf_W, linf_b):
    src = edge_index[0]
    dst = edge_index[1]
    pad = (N + jnp.arange(EP - E, dtype=jnp.int32) % (NP - N)).astype(jnp.int32)
    src_p = jnp.concatenate([src, pad])
    dst_p = jnp.concatenate([dst, pad])
    src_w = src_p.reshape(NW, CHUNKS, K)
    dst_w = dst_p.reshape(NW, CHUNKS, K)
    x_p = jnp.concatenate([x, jnp.zeros((NP - N, x.shape[1]), _f32)])

    degp = _deg_call(dst_p)

    g1 = pl.pallas_call(
        _s1_body,
        out_shape=jax.ShapeDtypeStruct((NP, D), _f32),
    )(x_p, W1, degp)

    agg1 = _agg_call(g1, src_w, dst_w)

    g2, hw2 = pl.pallas_call(
        _s2_body,
        out_shape=[jax.ShapeDtypeStruct((NP, D), _f32),
                   jax.ShapeDtypeStruct((NP, D), _f32)],
    )(agg1, x_p, W1, b1.reshape(1, D), W2, degp)

    agg2 = _agg_call(g2, src_w, dst_w)

    A, B = pl.pallas_call(
        _s3_body,
        out_shape=[jax.ShapeDtypeStruct((NP, D), _f32),
                   jax.ShapeDtypeStruct((NP, D), _f32)],
    )(agg2, hw2, b2.reshape(1, D), lin1_W[:D], lin1_W[D:], lin1_b.reshape(1, D),
      degp)

    s = _head_call(A, B, src_w, dst_w)

    outp = pl.pallas_call(
        _s4_body,
        grid=(EP // BE,),
        in_specs=[
            pl.BlockSpec((BE, D), lambda i: (i, 0)),
            pl.BlockSpec((D, C), lambda i: (0, 0)),
            pl.BlockSpec((1, C), lambda i: (0, 0)),
        ],
        out_specs=pl.BlockSpec((BE, C), lambda i: (i, 0)),
        out_shape=jax.ShapeDtypeStruct((EP, C), _f32),
    )(s, linf_W, linf_b.reshape(1, C))

    return lax.slice(outp, (0, 0), (E, C))


# speedup vs baseline: 2.7626x; 1.1219x over previous
import functools

import jax
import jax.numpy as jnp
from jax import lax
from jax.experimental import pallas as pl
from jax.experimental.pallas import tpu as pltpu
from jax.experimental.pallas import tpu_sc as plsc

N = 10000
NP = 10240
D = 128
E = 640000
C = 2
K = 128
NC, NS = 2, 16
NW = NC * NS
CB = 16
NB = 10
CHUNKS = CB * NB
EP = CHUNKS * NW * K
EPW = CHUNKS * K
RPT = NP // NS

_mesh = plsc.VectorSubcoreMesh(core_axis_name="c", subcore_axis_name="s",
                               num_cores=NC, num_subcores=NS)
_f32 = jnp.float32


def _fill2d(ref, rows, cols, val):
    def row(i, _):
        def col(j, _):
            ref[i, pl.ds(j * 16, 16)] = jnp.full((16,), val, _f32)
            return 0
        return lax.fori_loop(0, cols // 16, col, 0)
    lax.fori_loop(0, rows, row, 0)


def _deg_body(dst_hbm, out_hbm, didx_v, loc_v, red_v, part_sh, semi):
    c = lax.axis_index("c")
    s = lax.axis_index("s")
    w = s * NC + c
    cp = pltpu.async_copy(dst_hbm.at[w], didx_v, semi)

    def z(i, _):
        loc_v[pl.ds(i * 16, 16)] = jnp.zeros((16,), _f32)
        return 0
    lax.fori_loop(0, NP // 16, z, 0)
    cp.wait()
    ones = jnp.ones((16,), _f32)

    def ch(j, _):
        for g in range(K // 16):
            idx = didx_v[j, pl.ds(g * 16, 16)]
            plsc.addupdate_scatter(loc_v, [idx], ones)
        return 0
    lax.fori_loop(0, CHUNKS, ch, 0)
    pltpu.sync_copy(loc_v, part_sh.at[s])
    plsc.subcore_barrier()
    for t in range(NS):
        pltpu.sync_copy(part_sh.at[t, pl.ds(s * RPT, RPT)], red_v.at[t])

    def r(jj, _):
        sl = pl.ds(jj * 16, 16)
        acc = jnp.zeros((16,), _f32)
        for t in range(NS):
            acc = acc + red_v[t, sl]
        loc_v[sl] = acc
        return 0
    lax.fori_loop(0, RPT // 16, r, 0)
    pltpu.sync_copy(loc_v.at[pl.ds(0, RPT)], out_hbm.at[c, pl.ds(s * RPT, RPT)])


_deg_call = functools.partial(
    pl.kernel,
    out_type=jax.ShapeDtypeStruct((NC, NP), _f32),
    mesh=_mesh,
    compiler_params=pltpu.CompilerParams(needs_layout_passes=False),
    scratch_types=[
        pltpu.VMEM((CHUNKS, K), jnp.int32),
        pltpu.VMEM((NP,), _f32),
        pltpu.VMEM((NS, RPT), _f32),
        pltpu.VMEM_SHARED((NS, NP), _f32),
        pltpu.SemaphoreType.DMA,
    ],
)(_deg_body)


def _agg_body(g_hbm, src_hbm, dst_hbm, out_hbm,
              sidx_v, didx_v, rows0_v, rows1_v, zb_v, acc_sh,
              sem0, sem1):
    c = lax.axis_index("c")
    s = lax.axis_index("s")
    w = s * NC + c
    _fill2d(zb_v, 32, D, 0.0)

    def z(i, _):
        pltpu.sync_copy(zb_v, acc_sh.at[pl.ds(s * RPT + i * 32, 32)])
        return 0
    lax.fori_loop(0, RPT // 32, z, 0)
    plsc.subcore_barrier()

    def gstart(j, rv, sem):
        pltpu.async_copy(g_hbm.at[sidx_v.at[j]], rv, sem)

    def gwait(j, rv, sem):
        pltpu.make_async_copy(g_hbm.at[sidx_v.at[j]], rv, sem).wait()

    def scat(j, rv):
        pltpu.sync_copy(rv, acc_sh.at[didx_v.at[j]], add=True)

    def block(b, _):
        pltpu.sync_copy(src_hbm.at[w, pl.ds(b * CB, CB)], sidx_v)
        pltpu.sync_copy(dst_hbm.at[w, pl.ds(b * CB, CB)], didx_v)
        gstart(0, rows0_v, sem0)

        def pair(t, _):
            j0 = 2 * t
            gstart(j0 + 1, rows1_v, sem1)
            gwait(j0, rows0_v, sem0)
            scat(j0, rows0_v)
            gstart(j0 + 2, rows0_v, sem0)
            gwait(j0 + 1, rows1_v, sem1)
            scat(j0 + 1, rows1_v)
            return 0
        lax.fori_loop(0, CB // 2 - 1, pair, 0)
        gstart(CB - 1, rows1_v, sem1)
        gwait(CB - 2, rows0_v, sem0)
        scat(CB - 2, rows0_v)
        gwait(CB - 1, rows1_v, sem1)
        scat(CB - 1, rows1_v)
        return 0
    lax.fori_loop(0, NB, block, 0)
    plsc.subcore_barrier()
    pltpu.sync_copy(acc_sh.at[pl.ds(s * RPT, RPT)],
                    out_hbm.at[c, pl.ds(s * RPT, RPT)])


_agg_call = functools.partial(
    pl.kernel,
    out_type=jax.ShapeDtypeStruct((NC, NP, D), _f32),
    mesh=_mesh,
    scratch_types=[
        pltpu.VMEM((CB, K), jnp.int32),
        pltpu.VMEM((CB, K), jnp.int32),
        pltpu.VMEM((K, D), _f32),
        pltpu.VMEM((K, D), _f32),
        pltpu.VMEM((32, D), _f32),
        pltpu.VMEM_SHARED((NP, D), _f32),
        pltpu.SemaphoreType.DMA,
        pltpu.SemaphoreType.DMA,
    ],
)(_agg_body)


def _head_body(a_hbm, b_hbm, src_hbm, dst_hbm, out_hbm,
               sidx_v, didx_v, ra0_v, ra1_v, rb0_v, rb1_v, t0_v, t1_v,
               semg0, semg1, semo0, semo1):
    c = lax.axis_index("c")
    s = lax.axis_index("s")
    w = s * NC + c
    semg = (semg0, semg1)
    semo = (semo0, semo1)
    ras = (ra0_v, ra1_v)
    rbs = (rb0_v, rb1_v)
    ts = (t0_v, t1_v)

    def gstart(j, p):
        pltpu.async_copy(a_hbm.at[sidx_v.at[j]], ras[p], semg[p])
        pltpu.async_copy(b_hbm.at[didx_v.at[j]], rbs[p], semg[p])

    def gwait(j, p):
        pltpu.make_async_copy(a_hbm.at[sidx_v.at[j]], ras[p], semg[p]).wait()
        pltpu.make_async_copy(b_hbm.at[didx_v.at[j]], rbs[p], semg[p]).wait()

    def combine(p):
        def row(i, _):
            for g in range(D // 16):
                sl = pl.ds(g * 16, 16)
                ts[p][i, sl] = jnp.maximum(ras[p][i, sl] + rbs[p][i, sl], 0.0)
            return 0
        lax.fori_loop(0, K, row, 0)

    def ostart(p, base):
        pltpu.async_copy(ts[p], out_hbm.at[pl.ds(base, K)], semo[p])

    def owait(p):
        pltpu.make_async_copy(ts[p], out_hbm.at[pl.ds(w * EPW, K)],
                              semo[p]).wait()

    def block(b, _):
        pltpu.sync_copy(src_hbm.at[w, pl.ds(b * CB, CB)], sidx_v)
        pltpu.sync_copy(dst_hbm.at[w, pl.ds(b * CB, CB)], didx_v)
        gstart(0, 0)
        gstart(1, 1)

        def pair(t, _):
            for dp in (0, 1):
                j = 2 * t + dp
                gwait(j, dp)

                @pl.when((t > 0) | (b > 0))
                def _(dp=dp):
                    owait(dp)
                combine(dp)
                ostart(dp, w * EPW + (b * CB + j) * K)

                @pl.when(j + 2 < CB)
                def _(j=j, dp=dp):
                    gstart(j + 2, dp)
            return 0
        lax.fori_loop(0, CB // 2, pair, 0)
        return 0
    lax.fori_loop(0, NB, block, 0)
    owait(0)
    owait(1)


_head_call = functools.partial(
    pl.kernel,
    out_type=jax.ShapeDtypeStruct((EP, D), _f32),
    mesh=_mesh,
    scratch_types=[
        pltpu.VMEM((CB, K), jnp.int32),
        pltpu.VMEM((CB, K), jnp.int32),
        pltpu.VMEM((K, D), _f32),
        pltpu.VMEM((K, D), _f32),
        pltpu.VMEM((K, D), _f32),
        pltpu.VMEM((K, D), _f32),
        pltpu.VMEM((K, D), _f32),
        pltpu.VMEM((K, D), _f32),
        pltpu.SemaphoreType.DMA,
        pltpu.SemaphoreType.DMA,
        pltpu.SemaphoreType.DMA,
        pltpu.SemaphoreType.DMA,
    ],
)(_head_body)


def _dis(degp_ref):
    deg = degp_ref[0, :] + degp_ref[1, :] + 1.0
    return lax.rsqrt(deg)[:, None]


def _s1_body(x_ref, w1_ref, degp_ref, g1_ref):
    dis = _dis(degp_ref)
    hw = jnp.dot(x_ref[...], w1_ref[...], preferred_element_type=_f32)
    g1_ref[...] = hw * dis


def _s2_body(agg_ref, x_ref, w1_ref, b1_ref, w2_ref, degp_ref, g2_ref, hw2_ref):
    dis = _dis(degp_ref)
    hw1 = jnp.dot(x_ref[...], w1_ref[...], preferred_element_type=_f32)
    h1 = jnp.maximum(dis * (agg_ref[0] + agg_ref[1]) + dis * dis * hw1
                     + b1_ref[...], 0.0)
    hw2 = jnp.dot(h1, w2_ref[...], preferred_element_type=_f32)
    hw2_ref[...] = hw2
    g2_ref[...] = hw2 * dis


def _s3_body(agg_ref, hw2_ref, b2_ref, w1a_ref, w1b_ref, l1b_ref, degp_ref,
             a_ref, b_ref):
    dis = _dis(degp_ref)
    h2 = jnp.maximum(dis * (agg_ref[0] + agg_ref[1]) + dis * dis * hw2_ref[...]
                     + b2_ref[...], 0.0)
    a_ref[...] = jnp.dot(h2, w1a_ref[...], preferred_element_type=_f32) + l1b_ref[...]
    b_ref[...] = jnp.dot(h2, w1b_ref[...], preferred_element_type=_f32)


BE = 4096


def _s4_body(s_ref, wf_ref, bf_ref, o_ref):
    t = s_ref[...]
    z = jnp.dot(t, wf_ref[...], preferred_element_type=_f32) + bf_ref[...]
    m = jnp.max(z, axis=1, keepdims=True)
    o_ref[...] = z - m - jnp.log(jnp.sum(jnp.exp(z - m), axis=1, keepdims=True))


def kernel(x, edge_index, W1, b1, W2, b2, lin1_W, lin1_b, linf_W, linf_b):
    src = edge_index[0]
    dst = edge_index[1]
    pad = (N + jnp.arange(EP - E, dtype=jnp.int32) % (NP - N)).astype(jnp.int32)
    src_p = jnp.concatenate([src, pad])
    dst_p = jnp.concatenate([dst, pad])
    src_w = src_p.reshape(NW, CHUNKS, K)
    dst_w = dst_p.reshape(NW, CHUNKS, K)
    x_p = jnp.concatenate([x, jnp.zeros((NP - N, x.shape[1]), _f32)])

    degp = _deg_call(dst_w)

    g1 = pl.pallas_call(
        _s1_body,
        out_shape=jax.ShapeDtypeStruct((NP, D), _f32),
    )(x_p, W1, degp)

    agg1 = _agg_call(g1, src_w, dst_w)

    g2, hw2 = pl.pallas_call(
        _s2_body,
        out_shape=[jax.ShapeDtypeStruct((NP, D), _f32),
                   jax.ShapeDtypeStruct((NP, D), _f32)],
    )(agg1, x_p, W1, b1.reshape(1, D), W2, degp)

    agg2 = _agg_call(g2, src_w, dst_w)

    A, B = pl.pallas_call(
        _s3_body,
        out_shape=[jax.ShapeDtypeStruct((NP, D), _f32),
                   jax.ShapeDtypeStruct((NP, D), _f32)],
    )(agg2, hw2, b2.reshape(1, D), lin1_W[:D], lin1_W[D:], lin1_b.reshape(1, D),
      degp)

    s = _head_call(A, B, src_w, dst_w)

    outp = pl.pallas_call(
        _s4_body,
        grid=(EP // BE,),
        in_specs=[
            pl.BlockSpec((BE, D), lambda i: (i, 0)),
            pl.BlockSpec((D, C), lambda i: (0, 0)),
            pl.BlockSpec((1, C), lambda i: (0, 0)),
        ],
        out_specs=pl.BlockSpec((BE, C), lambda i: (i, 0)),
        out_shape=jax.ShapeDtypeStruct((EP, C), _f32),
    )(s, linf_W, linf_b.reshape(1, C))

    return lax.slice(outp, (0, 0), (E, C))
